# baseline (speedup 1.0000x reference)
import functools

import jax
import jax.numpy as jnp
from jax import lax
from jax.experimental import pallas as pl
from jax.experimental.pallas import tpu as pltpu
from jax.experimental.pallas import tpu_sc as plsc

VOCAB = 1000000
DIM = 32
B = 16384
NCOLS = 5
RPS = 4

_info = plsc.get_sparse_core_info()
NC, NS, L = _info.num_cores, _info.num_subcores, _info.num_lanes
NW = NC * NS
BPW = B // NW
CHUNKS = 4
CB = BPW // CHUNKS

_VB = 4096
R = 253952
_NVB = R // _VB
_MAXBLK = VOCAB // _VB


def _conv_body(x0_ref, x1_ref, x2_ref, x3_ref, y_ref):
    acc = None
    for q, x_ref in enumerate((x0_ref, x1_ref, x2_ref, x3_ref)):
        eye = jnp.eye(DIM, RPS * DIM, k=q * DIM, dtype=jnp.float32)
        part = jax.lax.dot_general(
            x_ref[...], eye, (((0,), (0,)), ((), ())),
            precision=jax.lax.Precision.HIGHEST)
        acc = part if acc is None else acc + part
    y_ref[...] = acc


def _make_in_spec(q):
    return pl.BlockSpec(
        (DIM, _VB), lambda i, q=q: (0, jnp.minimum(q * _NVB + i, _MAXBLK)))


def _convert(table_t):
    return pl.pallas_call(
        _conv_body,
        grid=(_NVB,),
        in_specs=[_make_in_spec(q) for q in range(RPS)],
        out_specs=pl.BlockSpec((_VB, RPS * DIM), lambda i: (i, 0)),
        out_shape=jax.ShapeDtypeStruct((R, RPS * DIM), jnp.float32),
    )(table_t, table_t, table_t, table_t)


def _quarter(v):
    one = jnp.int32(1)
    zero = jnp.int32(0)
    q = jnp.where(v >= R, one, zero)
    q = q + jnp.where(v >= 2 * R, one, zero)
    q = q + jnp.where(v >= 3 * R, one, zero)
    return q


def _body(tgt_hbm, ctx_hbm, ttab_hbm, ctab_hbm, out_hbm,
          tidx, cidx, tslab, cslab, wrows, crows, outv, sem1, sem2):
    wid = lax.axis_index("s") * NC + lax.axis_index("c")
    base = wid * BPW

    pltpu.sync_copy(tgt_hbm.at[pl.ds(base, BPW)], tidx)
    pltpu.sync_copy(ctx_hbm.at[pl.ds(base * NCOLS, BPW * NCOLS)], cidx)

    lane = lax.broadcasted_iota(jnp.int32, (L,), 0)
    for i in range(BPW // L):
        v = tidx[pl.ds(i * L, L)]
        tslab[pl.ds(i * L, L)] = v - _quarter(v) * R
    for i in range(BPW * NCOLS // L):
        v = cidx[pl.ds(i * L, L)]
        cslab[pl.ds(i * L, L)] = v - _quarter(v) * R

    def chunk_body(c, carry):
        cp1 = pltpu.async_copy(
            ttab_hbm.at[tslab.at[pl.ds(c * CB, CB)]], wrows, sem1)
        cp2 = pltpu.async_copy(
            ctab_hbm.at[cslab.at[pl.ds(c * CB * NCOLS, CB * NCOLS)]], crows, sem2)
        cp1.wait()
        cp2.wait()

        for blk in range(CB // L):
            row = blk * L + lane
            tv = tidx[pl.ds(c * CB + blk * L, L)]
            woff = _quarter(tv) * DIM
            crow = [row * NCOLS + n for n in range(NCOLS)]
            coff = []
            for n in range(NCOLS):
                cv = plsc.load_gather(cidx, [c * CB * NCOLS + crow[n]])
                coff.append(_quarter(cv) * DIM)
            accs = [jnp.zeros((L,), jnp.float32) for _ in range(NCOLS)]
            for d in range(DIM):
                w = plsc.load_gather(wrows, [row, woff + d])
                for n in range(NCOLS):
                    cc = plsc.load_gather(crows, [crow[n], coff[n] + d])
                    accs[n] = accs[n] + cc * w
            for n in range(NCOLS):
                outv[pl.ds(n * BPW + c * CB + blk * L, L)] = accs[n]
        return carry

    lax.fori_loop(0, CHUNKS, chunk_body, 0)

    for n in range(NCOLS):
        pltpu.sync_copy(outv.at[pl.ds(n * BPW, BPW)],
                        out_hbm.at[pl.ds(n * B + base, BPW)])


@jax.jit
def kernel(target, context, target_table, context_table):
    tgt = target.reshape(B)
    ctx = context.reshape(B * NCOLS)
    ttab = _convert(target_table.T)
    ctab = _convert(context_table.T)
    mesh = plsc.VectorSubcoreMesh(core_axis_name="c", subcore_axis_name="s")
    k = functools.partial(
        pl.kernel,
        mesh=mesh,
        out_type=jax.ShapeDtypeStruct((NCOLS * B,), jnp.float32),
        scratch_types=[
            pltpu.VMEM((BPW,), jnp.int32),
            pltpu.VMEM((BPW * NCOLS,), jnp.int32),
            pltpu.VMEM((BPW,), jnp.int32),
            pltpu.VMEM((BPW * NCOLS,), jnp.int32),
            pltpu.VMEM((CB, RPS * DIM), jnp.float32),
            pltpu.VMEM((CB * NCOLS, RPS * DIM), jnp.float32),
            pltpu.VMEM((NCOLS * BPW,), jnp.float32),
            pltpu.SemaphoreType.DMA,
            pltpu.SemaphoreType.DMA,
        ],
        compiler_params=pltpu.CompilerParams(
            needs_layout_passes=False, use_tc_tiling_on_sc=True),
    )(_body)
    out = k(tgt, ctx, ttab, ctab)
    return out.reshape(NCOLS, B).T


# speedup vs baseline: 1.8927x; 1.8927x over previous
import functools

import jax
import jax.numpy as jnp
from jax import lax
from jax.experimental import pallas as pl
from jax.experimental.pallas import tpu as pltpu
from jax.experimental.pallas import tpu_sc as plsc

VOCAB = 1000000
DIM = 32
B = 16384
NCOLS = 5
RPS = 4

_info = plsc.get_sparse_core_info()
NC, NS, L = _info.num_cores, _info.num_subcores, _info.num_lanes
NW = NC * NS
BPW = B // NW
CHUNKS = 4
CB = BPW // CHUNKS

_VB = 8192
R = 253952
_NVB = R // _VB
_MAXBLK = VOCAB // _VB


def _conv_body(x0_ref, x1_ref, x2_ref, x3_ref, y_ref):
    y_ref[...] = jnp.concatenate(
        [x0_ref[...].T, x1_ref[...].T, x2_ref[...].T, x3_ref[...].T], axis=1)


def _make_in_spec(q):
    return pl.BlockSpec(
        (DIM, _VB), lambda i, q=q: (0, jnp.minimum(q * _NVB + i, _MAXBLK)))


def _convert(table_t):
    return pl.pallas_call(
        _conv_body,
        grid=(_NVB,),
        in_specs=[_make_in_spec(q) for q in range(RPS)],
        out_specs=pl.BlockSpec((_VB, RPS * DIM), lambda i: (i, 0)),
        out_shape=jax.ShapeDtypeStruct((R, RPS * DIM), jnp.float32),
    )(table_t, table_t, table_t, table_t)


def _quarter(v):
    one = jnp.int32(1)
    zero = jnp.int32(0)
    q = jnp.where(v >= R, one, zero)
    q = q + jnp.where(v >= 2 * R, one, zero)
    q = q + jnp.where(v >= 3 * R, one, zero)
    return q


def _body(tgt_hbm, ctx_hbm, ttab_hbm, ctab_hbm, out_hbm,
          tidx, cidx, tslab, cslab, wrows, crows, outv, sem1, sem2):
    wid = lax.axis_index("s") * NC + lax.axis_index("c")
    base = wid * BPW

    pltpu.sync_copy(tgt_hbm.at[pl.ds(base, BPW)], tidx)
    pltpu.sync_copy(ctx_hbm.at[pl.ds(base * NCOLS, BPW * NCOLS)], cidx)

    lane = lax.broadcasted_iota(jnp.int32, (L,), 0)
    for i in range(BPW // L):
        v = tidx[pl.ds(i * L, L)]
        tslab[pl.ds(i * L, L)] = v - _quarter(v) * R
    for i in range(BPW * NCOLS // L):
        v = cidx[pl.ds(i * L, L)]
        cslab[pl.ds(i * L, L)] = v - _quarter(v) * R

    def chunk_body(c, carry):
        cp1 = pltpu.async_copy(
            ttab_hbm.at[tslab.at[pl.ds(c * CB, CB)]], wrows, sem1)
        cp2 = pltpu.async_copy(
            ctab_hbm.at[cslab.at[pl.ds(c * CB * NCOLS, CB * NCOLS)]], crows, sem2)
        cp1.wait()
        cp2.wait()

        for blk in range(CB // L):
            row = blk * L + lane
            tv = tidx[pl.ds(c * CB + blk * L, L)]
            woff = _quarter(tv) * DIM
            crow = [row * NCOLS + n for n in range(NCOLS)]
            coff = []
            for n in range(NCOLS):
                cv = plsc.load_gather(cidx, [c * CB * NCOLS + crow[n]])
                coff.append(_quarter(cv) * DIM)
            accs = [jnp.zeros((L,), jnp.float32) for _ in range(NCOLS)]
            for d in range(DIM):
                w = plsc.load_gather(wrows, [row, woff + d])
                for n in range(NCOLS):
                    cc = plsc.load_gather(crows, [crow[n], coff[n] + d])
                    accs[n] = accs[n] + cc * w
            for n in range(NCOLS):
                outv[pl.ds(n * BPW + c * CB + blk * L, L)] = accs[n]
        return carry

    lax.fori_loop(0, CHUNKS, chunk_body, 0)

    for n in range(NCOLS):
        pltpu.sync_copy(outv.at[pl.ds(n * BPW, BPW)],
                        out_hbm.at[pl.ds(n * B + base, BPW)])


@jax.jit
def kernel(target, context, target_table, context_table):
    tgt = target.reshape(B)
    ctx = context.reshape(B * NCOLS)
    ttab = _convert(target_table.T)
    ctab = _convert(context_table.T)
    mesh = plsc.VectorSubcoreMesh(core_axis_name="c", subcore_axis_name="s")
    k = functools.partial(
        pl.kernel,
        mesh=mesh,
        out_type=jax.ShapeDtypeStruct((NCOLS * B,), jnp.float32),
        scratch_types=[
            pltpu.VMEM((BPW,), jnp.int32),
            pltpu.VMEM((BPW * NCOLS,), jnp.int32),
            pltpu.VMEM((BPW,), jnp.int32),
            pltpu.VMEM((BPW * NCOLS,), jnp.int32),
            pltpu.VMEM((CB, RPS * DIM), jnp.float32),
            pltpu.VMEM((CB * NCOLS, RPS * DIM), jnp.float32),
            pltpu.VMEM((NCOLS * BPW,), jnp.float32),
            pltpu.SemaphoreType.DMA,
            pltpu.SemaphoreType.DMA,
        ],
        compiler_params=pltpu.CompilerParams(
            needs_layout_passes=False, use_tc_tiling_on_sc=True),
    )(_body)
    out = k(tgt, ctx, ttab, ctab)
    return out.reshape(NCOLS, B).T
